# baseline (speedup 1.0000x reference)
import jax
import jax.numpy as jnp
from jax.experimental import pallas as pl
from jax.experimental.pallas import tpu as pltpu

OBJ_N = 128
WALL_N = 8
TOTAL = 137
NP = 144
D = 512
STEPS = 4
INV_DEG = 1.0 / float(TOTAL - 1)


def _mm(x, w):
    return jax.lax.dot_general(x, w, (((1,), (0,)), ((), ())),
                               preferred_element_type=jnp.float32)


def _mmb(x, w):
    return jax.lax.dot_general(x.astype(jnp.bfloat16), w,
                               (((1,), (0,)), ((), ())),
                               preferred_element_type=jnp.float32)


def _relu(x):
    return jnp.maximum(x, 0.0)


def _body(obj_x, rel_at, rel_bt, wall_x, floor_x,
          w_obj1, b_obj1, w_obj2, b_obj2,
          w_rel1, b_rel1, w_rel2, b_rel2,
          w_wal1, b_wal1, w_wal2, b_wal2,
          w_flr1, b_flr1, w_flr2, b_flr2,
          w_nn, b_nn, w_sp, b_sp, w_op, b_op, w_un, b_un,
          w_pspo, b_pspo, w_ur, b_ur,
          w_d1, b_d1, w_d2, b_d2,
          out,
          rels, nodes, acc_s, acc_o, abuf, bbuf):
    f32 = jnp.float32

    h = _relu(_mm(obj_x[...], w_obj1[...]) + b_obj1[...])
    obj_emb = _mm(h, w_obj2[...]) + b_obj2[...]
    h = _relu(_mm(wall_x[...], w_wal1[...]) + b_wal1[...])
    wall_emb = _mm(h, w_wal2[...]) + b_wal2[...]
    h = _relu(_mm(floor_x[...], w_flr1[...]) + b_flr1[...])
    floor_emb = _mm(h, w_flr2[...]) + b_flr2[...]
    nodes[0:OBJ_N, :] = obj_emb
    nodes[OBJ_N:OBJ_N + WALL_N, :] = wall_emb
    nodes[TOTAL - 1:TOTAL, :] = floor_emb[0:1, :]
    nodes[TOTAL:NP, :] = jnp.zeros((NP - TOTAL, D), f32)

    acc_o[...] = jnp.zeros((NP, D), f32)
    acc_s[...] = jnp.zeros((NP, D), f32)

    w0 = w_rel1[0:1, :]
    w1 = w_rel1[1:2, :]
    br1 = b_rel1[...]
    br2 = b_rel2[...]

    BC = 8
    ohbase = (jax.lax.broadcasted_iota(jnp.int32, (OBJ_N, BC), 0)
              - jax.lax.broadcasted_iota(jnp.int32, (OBJ_N, BC), 1))
    subj_m = jax.lax.broadcasted_iota(jnp.int32, (BC, OBJ_N, D), 0)
    j_m = jax.lax.broadcasted_iota(jnp.int32, (BC, OBJ_N, D), 1)
    subj_t = jax.lax.broadcasted_iota(jnp.int32, (BC, NP - OBJ_N, D), 0)
    j_t = jax.lax.broadcasted_iota(jnp.int32, (BC, NP - OBJ_N, D), 1) + OBJ_N

    def build(c, _):
        base = c * BC
        oh = (ohbase == base).astype(f32)
        a_g = _mm(rel_at[...], oh)
        b_g = _mm(rel_bt[...], oh)
        a_st = jnp.concatenate([a_g[:, m:m + 1] for m in range(BC)], axis=0)
        b_st = jnp.concatenate([b_g[:, m:m + 1] for m in range(BC)], axis=0)
        hh = _relu(a_st * w0 + b_st * w1 + br1)
        emb = _mm(hh.astype(jnp.bfloat16), w_rel2[...]) + br2
        emb3 = emb.reshape(BC, OBJ_N, D)
        sm = subj_m + base
        st = subj_t + base
        main3 = jnp.where((j_m != sm) & (sm < TOTAL),
                          jnp.where(sm < OBJ_N, emb3, 0.001), 0.0)
        tail3 = jnp.where((j_t < TOTAL) & (j_t != st) & (st < TOTAL),
                          0.001, 0.0)
        full3 = jnp.concatenate([main3, tail3], axis=1)
        rels[pl.ds(base, BC)] = full3.astype(jnp.bfloat16)
        acc_s[pl.ds(base, BC), :] = jnp.sum(full3, axis=1)
        acc_o[...] += jnp.sum(full3, axis=0)
        return 0

    jax.lax.fori_loop(0, NP // BC, build, 0)

    row_iota = jax.lax.broadcasted_iota(jnp.int32, (NP, D), 0)
    SC = 8
    dg = jnp.zeros((NP, D), f32)
    pc = jnp.zeros((NP, D), f32)
    pr = jnp.zeros((NP, D), f32)
    for t in range(STEPS):
        nodes_v = nodes[...]
        mean = jnp.sum(nodes_v, axis=0, keepdims=True) / float(TOTAL)
        c_nn = _relu(_mmb(mean, w_nn[...]) + b_nn[...])
        agg_s = (acc_s[...] - dg - 7.0 * pc) * INV_DEG
        agg_o = (acc_o[...] - dg - 7.0 * pr) * INV_DEG
        c_sp = _relu(_mmb(agg_s, w_sp[...]) + b_sp[...])
        c_op = _relu(_mmb(agg_o, w_op[...]) + b_op[...])
        c = (c_nn + c_sp + c_op) / 3.0
        new_nodes = _relu(nodes_v + _mmb(c, w_un[...]) + b_un[...])
        new_nodes = jnp.where(row_iota < TOTAL, new_nodes, 0.0)
        nodes[...] = new_nodes

        if t < STEPS - 1:
            hcat = _relu(_mmb(new_nodes, w_pspo[...]) + b_pspo[...])
            hst = jnp.concatenate([hcat[:, 0:D], hcat[:, D:2 * D]], axis=0)
            ab = _mmb(hst, w_ur[...]) * 0.5
            abuf[...] = ab[0:NP] + b_ur[...]
            bbuf[...] = ab[NP:2 * NP]
            a_full = abuf[...]
            b_full = bbuf[...]
            dg = _relu(dg + a_full + b_full)
            pc = _relu(pc + a_full + b_full[TOTAL:TOTAL + 1, :])
            pr = _relu(pr + a_full[TOTAL:TOTAL + 1, :] + b_full)
            acc_o[...] = jnp.zeros((NP, D), f32)
            last = t == STEPS - 2

            def sweep(c, _):
                base = c * SC
                blk = rels[pl.ds(base, SC)].astype(f32)
                a3 = abuf[pl.ds(base, SC), :].reshape(SC, 1, D)
                b3 = bbuf[...].reshape(1, NP, D)
                new = _relu(blk + a3 + b3)
                if not last:
                    rels[pl.ds(base, SC)] = new.astype(jnp.bfloat16)
                acc_s[pl.ds(base, SC), :] = jnp.sum(new, axis=1)
                acc_o[...] += jnp.sum(new, axis=0)
                return 0

            jax.lax.fori_loop(0, NP // SC, sweep, 0)

    obj = nodes[0:OBJ_N, :]
    hh = _mm(obj, w_d1[...]) + b_d1[...]
    hh = jnp.where(hh > 0, hh, 0.2 * hh)
    out[...] = _mm(hh, w_d2[...]) + b_d2[...]


def kernel(trans_object_obb, trans_object_abb, trans_object_obb_center,
           trans_object_obb_center_dist, trans_object_abb_eiou,
           wall_position, wall_normal, floor_position, floor_normal,
           floor_z_value, params):
    f32 = jnp.float32

    obj_in = jnp.concatenate([trans_object_obb[0], trans_object_abb[0],
                              trans_object_obb_center[0]], -1)
    obj_x = jnp.pad(obj_in, ((0, 0), (0, 7)))
    w_obj1 = jnp.pad(params['obj1'][0], ((0, 7), (0, 0)))

    rel_at = trans_object_obb_center_dist[0].reshape(OBJ_N, OBJ_N).T
    rel_bt = trans_object_abb_eiou[0].reshape(OBJ_N, OBJ_N).T

    wall_in = jnp.concatenate([wall_position[0], wall_normal[0]], -1)
    wall_x = jnp.pad(wall_in, ((0, 0), (0, 1)))
    w_wal1 = jnp.pad(params['wal1'][0], ((0, 1), (0, 0)))

    floor_in = jnp.concatenate([floor_position[0], floor_normal[0],
                                floor_z_value[0]], -1)
    floor_x = jnp.pad(floor_in, ((0, 7), (0, 0)))

    b = lambda v: v.reshape(1, -1)
    bf16 = jnp.bfloat16

    w_pspo = jnp.concatenate([params['ps'][0], params['po'][0]],
                             axis=1).astype(bf16)
    b_pspo = jnp.concatenate([params['ps'][1], params['po'][1]],
                             axis=0).reshape(1, 2 * D)

    w_d1 = jnp.concatenate([params['t1'][0], params['e1'][0],
                            params['s1'][0]], axis=1)
    b_d1 = jnp.concatenate([params['t1'][1], params['e1'][1],
                            params['s1'][1]], axis=0).reshape(1, 768)
    h2 = D // 2
    w_d2 = jnp.zeros((3 * h2, 128), f32)
    w_d2 = w_d2.at[0:h2, 0:3].set(params['t2'][0])
    w_d2 = w_d2.at[h2:2 * h2, 3:6].set(params['e2'][0])
    w_d2 = w_d2.at[2 * h2:3 * h2, 6:9].set(params['s2'][0])
    b_d2 = jnp.zeros((1, 128), f32)
    b_d2 = b_d2.at[0, 0:3].set(params['t2'][1])
    b_d2 = b_d2.at[0, 3:6].set(params['e2'][1])
    b_d2 = b_d2.at[0, 6:9].set(params['s2'][1])

    args = [obj_x, rel_at, rel_bt, wall_x, floor_x,
            w_obj1, b(params['obj1'][1]), params['obj2'][0], b(params['obj2'][1]),
            params['rel1'][0], b(params['rel1'][1]),
            params['rel2'][0].astype(jnp.bfloat16), b(params['rel2'][1]),
            w_wal1, b(params['wal1'][1]), params['wal2'][0], b(params['wal2'][1]),
            params['flr1'][0], b(params['flr1'][1]), params['flr2'][0], b(params['flr2'][1]),
            params['nn'][0].astype(bf16), b(params['nn'][1]),
            params['sp'][0].astype(bf16), b(params['sp'][1]),
            params['op'][0].astype(bf16), b(params['op'][1]),
            params['un'][0].astype(bf16), b(params['un'][1]),
            w_pspo, b_pspo,
            params['ur'][0].astype(bf16), b(params['ur'][1]),
            w_d1, b_d1, w_d2, b_d2]

    out = pl.pallas_call(
        _body,
        out_shape=jax.ShapeDtypeStruct((OBJ_N, 128), f32),
        scratch_shapes=[
            pltpu.VMEM((NP, NP, D), jnp.bfloat16),
            pltpu.VMEM((NP, D), f32),
            pltpu.VMEM((NP, D), f32),
            pltpu.VMEM((NP, D), f32),
            pltpu.VMEM((NP, D), f32),
            pltpu.VMEM((NP, D), f32),
        ],
        compiler_params=pltpu.CompilerParams(
            vmem_limit_bytes=100 * 1024 * 1024),
    )(*args)
    return out[:, :9]


# speedup vs baseline: 1.0461x; 1.0461x over previous
import jax
import jax.numpy as jnp
from jax.experimental import pallas as pl
from jax.experimental.pallas import tpu as pltpu

OBJ_N = 128
WALL_N = 8
TOTAL = 137
NP = 144
D = 512
STEPS = 4
INV_DEG = 1.0 / float(TOTAL - 1)


def _mm(x, w):
    return jax.lax.dot_general(x, w, (((1,), (0,)), ((), ())),
                               preferred_element_type=jnp.float32)


def _mmb(x, w):
    return jax.lax.dot_general(x.astype(jnp.bfloat16), w,
                               (((1,), (0,)), ((), ())),
                               preferred_element_type=jnp.float32)


def _relu(x):
    return jnp.maximum(x, 0.0)


def _body(obj_x, rel_at, rel_bt, wall_x, floor_x,
          w_obj1, b_obj1, w_obj2, b_obj2,
          w_rel1, b_rel1, w_rel2, b_rel2,
          w_wal1, b_wal1, w_wal2, b_wal2,
          w_flr1, b_flr1, w_flr2, b_flr2,
          w_nn, b_nn, w_sp, b_sp, w_op, b_op, w_un, b_un,
          w_pspo, b_pspo, w_ur, b_ur,
          w_d1, b_d1, w_d2, b_d2,
          out,
          rels, nodes, acc_s, acc_o, abuf, bbuf):
    f32 = jnp.float32

    h = _relu(_mm(obj_x[...], w_obj1[...]) + b_obj1[...])
    obj_emb = _mm(h, w_obj2[...]) + b_obj2[...]
    h = _relu(_mm(wall_x[...], w_wal1[...]) + b_wal1[...])
    wall_emb = _mm(h, w_wal2[...]) + b_wal2[...]
    h = _relu(_mm(floor_x[...], w_flr1[...]) + b_flr1[...])
    floor_emb = _mm(h, w_flr2[...]) + b_flr2[...]
    nodes[0:OBJ_N, :] = obj_emb
    nodes[OBJ_N:OBJ_N + WALL_N, :] = wall_emb
    nodes[TOTAL - 1:TOTAL, :] = floor_emb[0:1, :]
    nodes[TOTAL:NP, :] = jnp.zeros((NP - TOTAL, D), f32)

    acc_o[...] = jnp.zeros((NP, D), f32)
    acc_s[...] = jnp.zeros((NP, D), f32)

    w0 = w_rel1[0:1, :]
    w1 = w_rel1[1:2, :]
    br1 = b_rel1[...]
    br2 = b_rel2[...]

    BC = 8
    ohbase = (jax.lax.broadcasted_iota(jnp.int32, (OBJ_N, BC), 0)
              - jax.lax.broadcasted_iota(jnp.int32, (OBJ_N, BC), 1))
    subj_m = jax.lax.broadcasted_iota(jnp.int32, (BC, OBJ_N, D), 0)
    j_m = jax.lax.broadcasted_iota(jnp.int32, (BC, OBJ_N, D), 1)
    subj_t = jax.lax.broadcasted_iota(jnp.int32, (BC, NP - OBJ_N, D), 0)
    j_t = jax.lax.broadcasted_iota(jnp.int32, (BC, NP - OBJ_N, D), 1) + OBJ_N

    def build(c, _):
        base = c * BC
        oh = (ohbase == base).astype(f32)
        a_g = _mm(rel_at[...], oh)
        b_g = _mm(rel_bt[...], oh)
        a_st = jnp.concatenate([a_g[:, m:m + 1] for m in range(BC)], axis=0)
        b_st = jnp.concatenate([b_g[:, m:m + 1] for m in range(BC)], axis=0)
        hh = _relu(a_st * w0 + b_st * w1 + br1)
        emb = _mm(hh.astype(jnp.bfloat16), w_rel2[...]) + br2
        emb3 = emb.reshape(BC, OBJ_N, D)
        sm = subj_m + base
        st = subj_t + base
        main3 = jnp.where((j_m != sm) & (sm < TOTAL),
                          jnp.where(sm < OBJ_N, emb3, 0.001), 0.0)
        tail3 = jnp.where((j_t < TOTAL) & (j_t != st) & (st < TOTAL),
                          0.001, 0.0)
        full3 = jnp.concatenate([main3, tail3], axis=1)
        rels[pl.ds(base, BC)] = full3
        acc_s[pl.ds(base, BC), :] = jnp.sum(full3, axis=1)
        acc_o[...] += jnp.sum(full3, axis=0)
        return 0

    jax.lax.fori_loop(0, NP // BC, build, 0)

    row_iota = jax.lax.broadcasted_iota(jnp.int32, (NP, D), 0)
    SC = 8
    dg = jnp.zeros((NP, D), f32)
    pc = jnp.zeros((NP, D), f32)
    pr = jnp.zeros((NP, D), f32)
    for t in range(STEPS):
        nodes_v = nodes[...]
        mean = jnp.sum(nodes_v, axis=0, keepdims=True) / float(TOTAL)
        c_nn = _relu(_mmb(mean, w_nn[...]) + b_nn[...])
        agg_s = (acc_s[...] - dg - 7.0 * pc) * INV_DEG
        agg_o = (acc_o[...] - dg - 7.0 * pr) * INV_DEG
        c_sp = _relu(_mmb(agg_s, w_sp[...]) + b_sp[...])
        c_op = _relu(_mmb(agg_o, w_op[...]) + b_op[...])
        c = (c_nn + c_sp + c_op) / 3.0
        new_nodes = _relu(nodes_v + _mmb(c, w_un[...]) + b_un[...])
        new_nodes = jnp.where(row_iota < TOTAL, new_nodes, 0.0)
        nodes[...] = new_nodes

        if t < STEPS - 1:
            hcat = _relu(_mmb(new_nodes, w_pspo[...]) + b_pspo[...])
            hst = jnp.concatenate([hcat[:, 0:D], hcat[:, D:2 * D]], axis=0)
            ab = _mmb(hst, w_ur[...]) * 0.5
            abuf[...] = ab[0:NP] + b_ur[...]
            bbuf[...] = ab[NP:2 * NP]
            a_full = abuf[...]
            b_full = bbuf[...]
            dg = _relu(dg + a_full + b_full)
            pc = _relu(pc + a_full + b_full[TOTAL:TOTAL + 1, :])
            pr = _relu(pr + a_full[TOTAL:TOTAL + 1, :] + b_full)
            acc_o[...] = jnp.zeros((NP, D), f32)
            last = t == STEPS - 2

            def sweep(c, _):
                base = c * SC
                blk = rels[pl.ds(base, SC)]
                a3 = abuf[pl.ds(base, SC), :].reshape(SC, 1, D)
                b3 = bbuf[...].reshape(1, NP, D)
                new = _relu(blk + a3 + b3)
                if not last:
                    rels[pl.ds(base, SC)] = new
                acc_s[pl.ds(base, SC), :] = jnp.sum(new, axis=1)
                acc_o[...] += jnp.sum(new, axis=0)
                return 0

            jax.lax.fori_loop(0, NP // SC, sweep, 0)

    obj = nodes[0:OBJ_N, :]
    hh = _mm(obj, w_d1[...]) + b_d1[...]
    hh = jnp.where(hh > 0, hh, 0.2 * hh)
    out[...] = _mm(hh, w_d2[...]) + b_d2[...]


def kernel(trans_object_obb, trans_object_abb, trans_object_obb_center,
           trans_object_obb_center_dist, trans_object_abb_eiou,
           wall_position, wall_normal, floor_position, floor_normal,
           floor_z_value, params):
    f32 = jnp.float32

    obj_in = jnp.concatenate([trans_object_obb[0], trans_object_abb[0],
                              trans_object_obb_center[0]], -1)
    obj_x = jnp.pad(obj_in, ((0, 0), (0, 7)))
    w_obj1 = jnp.pad(params['obj1'][0], ((0, 7), (0, 0)))

    rel_at = trans_object_obb_center_dist[0].reshape(OBJ_N, OBJ_N).T
    rel_bt = trans_object_abb_eiou[0].reshape(OBJ_N, OBJ_N).T

    wall_in = jnp.concatenate([wall_position[0], wall_normal[0]], -1)
    wall_x = jnp.pad(wall_in, ((0, 0), (0, 1)))
    w_wal1 = jnp.pad(params['wal1'][0], ((0, 1), (0, 0)))

    floor_in = jnp.concatenate([floor_position[0], floor_normal[0],
                                floor_z_value[0]], -1)
    floor_x = jnp.pad(floor_in, ((0, 7), (0, 0)))

    b = lambda v: v.reshape(1, -1)
    bf16 = jnp.bfloat16

    w_pspo = jnp.concatenate([params['ps'][0], params['po'][0]],
                             axis=1).astype(bf16)
    b_pspo = jnp.concatenate([params['ps'][1], params['po'][1]],
                             axis=0).reshape(1, 2 * D)

    w_d1 = jnp.concatenate([params['t1'][0], params['e1'][0],
                            params['s1'][0]], axis=1)
    b_d1 = jnp.concatenate([params['t1'][1], params['e1'][1],
                            params['s1'][1]], axis=0).reshape(1, 768)
    h2 = D // 2
    w_d2 = jnp.zeros((3 * h2, 128), f32)
    w_d2 = w_d2.at[0:h2, 0:3].set(params['t2'][0])
    w_d2 = w_d2.at[h2:2 * h2, 3:6].set(params['e2'][0])
    w_d2 = w_d2.at[2 * h2:3 * h2, 6:9].set(params['s2'][0])
    b_d2 = jnp.zeros((1, 128), f32)
    b_d2 = b_d2.at[0, 0:3].set(params['t2'][1])
    b_d2 = b_d2.at[0, 3:6].set(params['e2'][1])
    b_d2 = b_d2.at[0, 6:9].set(params['s2'][1])

    args = [obj_x, rel_at, rel_bt, wall_x, floor_x,
            w_obj1, b(params['obj1'][1]), params['obj2'][0], b(params['obj2'][1]),
            params['rel1'][0], b(params['rel1'][1]),
            params['rel2'][0].astype(jnp.bfloat16), b(params['rel2'][1]),
            w_wal1, b(params['wal1'][1]), params['wal2'][0], b(params['wal2'][1]),
            params['flr1'][0], b(params['flr1'][1]), params['flr2'][0], b(params['flr2'][1]),
            params['nn'][0].astype(bf16), b(params['nn'][1]),
            params['sp'][0].astype(bf16), b(params['sp'][1]),
            params['op'][0].astype(bf16), b(params['op'][1]),
            params['un'][0].astype(bf16), b(params['un'][1]),
            w_pspo, b_pspo,
            params['ur'][0].astype(bf16), b(params['ur'][1]),
            w_d1, b_d1, w_d2, b_d2]

    out = pl.pallas_call(
        _body,
        out_shape=jax.ShapeDtypeStruct((OBJ_N, 128), f32),
        scratch_shapes=[
            pltpu.VMEM((NP, NP, D), f32),
            pltpu.VMEM((NP, D), f32),
            pltpu.VMEM((NP, D), f32),
            pltpu.VMEM((NP, D), f32),
            pltpu.VMEM((NP, D), f32),
            pltpu.VMEM((NP, D), f32),
        ],
        compiler_params=pltpu.CompilerParams(
            vmem_limit_bytes=100 * 1024 * 1024),
    )(*args)
    return out[:, :9]


# speedup vs baseline: 1.4580x; 1.3937x over previous
import jax
import jax.numpy as jnp
from jax.experimental import pallas as pl
from jax.experimental.pallas import tpu as pltpu

OBJ_N = 128
WALL_N = 8
TOTAL = 137
NP = 144
D = 512
STEPS = 4
INV_DEG = 1.0 / float(TOTAL - 1)


def _mm(x, w):
    return jax.lax.dot_general(x, w, (((1,), (0,)), ((), ())),
                               preferred_element_type=jnp.float32)


def _relu(x):
    return jnp.maximum(x, 0.0)


def _body(obb, abb, ctr, rel_a, rel_b, wpos, wnrm, fpos, fnrm, fz,
          w_obj1, b_obj1, w_obj2, b_obj2,
          w_rel1, b_rel1, w_rel2, b_rel2,
          w_wal1, b_wal1, w_wal2, b_wal2,
          w_flr1, b_flr1, w_flr2, b_flr2,
          w_nn, b_nn, w_sp, b_sp, w_op, b_op, w_un, b_un,
          w_ps, b_ps, w_po, b_po, w_ur, b_ur,
          w_t1, b_t1, w_t2, b_t2,
          w_e1, b_e1, w_e2, b_e2,
          w_s1, b_s1, w_s2, b_s2,
          out,
          rels, nodes, acc_s, acc_o, abuf, bbuf, wrel2b):
    f32 = jnp.float32

    obj_x = jnp.concatenate([obb[...], abb[...], ctr[...]], axis=1)
    h = _relu(_mm(obj_x, w_obj1[...]) + b_obj1[...])
    obj_emb = _mm(h, w_obj2[...]) + b_obj2[...]
    wall_x = jnp.concatenate([wpos[...], wnrm[...]], axis=1)
    h = _relu(_mm(wall_x, w_wal1[...]) + b_wal1[...])
    wall_emb = _mm(h, w_wal2[...]) + b_wal2[...]
    floor_x = jnp.concatenate([fpos[...], fnrm[...], fz[...]], axis=1)
    h = _relu(_mm(floor_x, w_flr1[...]) + b_flr1[...])
    floor_emb = _mm(h, w_flr2[...]) + b_flr2[...]
    nodes[0:OBJ_N, :] = obj_emb
    nodes[OBJ_N:OBJ_N + WALL_N, :] = wall_emb
    nodes[TOTAL - 1:TOTAL, :] = floor_emb
    nodes[TOTAL:NP, :] = jnp.zeros((NP - TOTAL, D), f32)

    acc_o[...] = jnp.zeros((NP, D), f32)
    acc_s[...] = jnp.zeros((NP, D), f32)
    wrel2b[...] = w_rel2[...].astype(jnp.bfloat16)

    rat = jnp.transpose(rel_a[...], (1, 0))
    rbt = jnp.transpose(rel_b[...], (1, 0))
    w0 = w_rel1[0:1, :]
    w1 = w_rel1[1:2, :]
    br1 = b_rel1[...]
    br2 = b_rel2[...]

    BC = 8
    ohbase = (jax.lax.broadcasted_iota(jnp.int32, (OBJ_N, BC), 0)
              - jax.lax.broadcasted_iota(jnp.int32, (OBJ_N, BC), 1))
    subj_m = jax.lax.broadcasted_iota(jnp.int32, (BC, OBJ_N, D), 0)
    j_m = jax.lax.broadcasted_iota(jnp.int32, (BC, OBJ_N, D), 1)
    subj_t = jax.lax.broadcasted_iota(jnp.int32, (BC, NP - OBJ_N, D), 0)
    j_t = jax.lax.broadcasted_iota(jnp.int32, (BC, NP - OBJ_N, D), 1) + OBJ_N

    def build(c, _):
        base = c * BC
        oh = (ohbase == base).astype(f32)
        a_g = _mm(rat, oh)
        b_g = _mm(rbt, oh)
        a_st = jnp.concatenate([a_g[:, m:m + 1] for m in range(BC)], axis=0)
        b_st = jnp.concatenate([b_g[:, m:m + 1] for m in range(BC)], axis=0)
        hh = _relu(a_st * w0 + b_st * w1 + br1)
        emb = _mm(hh.astype(jnp.bfloat16), wrel2b[...]) + br2
        emb3 = emb.reshape(BC, OBJ_N, D)
        sm = subj_m + base
        st = subj_t + base
        main3 = jnp.where((j_m != sm) & (sm < TOTAL),
                          jnp.where(sm < OBJ_N, emb3, 0.001), 0.0)
        tail3 = jnp.where((j_t < TOTAL) & (j_t != st) & (st < TOTAL),
                          0.001, 0.0)
        full3 = jnp.concatenate([main3, tail3], axis=1)
        rels[pl.ds(base, BC)] = full3
        acc_s[pl.ds(base, BC), :] = jnp.sum(full3, axis=1)
        acc_o[...] += jnp.sum(full3, axis=0)
        return 0

    jax.lax.fori_loop(0, NP // BC, build, 0)

    row_iota = jax.lax.broadcasted_iota(jnp.int32, (NP, D), 0)
    SC = 8
    dg = jnp.zeros((NP, D), f32)
    pc = jnp.zeros((NP, D), f32)
    pr = jnp.zeros((NP, D), f32)
    for t in range(STEPS):
        nodes_v = nodes[...]
        mean = jnp.sum(nodes_v, axis=0, keepdims=True) / float(TOTAL)
        c_nn = _relu(_mm(mean, w_nn[...]) + b_nn[...])
        agg_s = (acc_s[...] - dg - 7.0 * pc) * INV_DEG
        agg_o = (acc_o[...] - dg - 7.0 * pr) * INV_DEG
        c_sp = _relu(_mm(agg_s, w_sp[...]) + b_sp[...])
        c_op = _relu(_mm(agg_o, w_op[...]) + b_op[...])
        c = (c_nn + c_sp + c_op) / 3.0
        new_nodes = _relu(nodes_v + _mm(c, w_un[...]) + b_un[...])
        new_nodes = jnp.where(row_iota < TOTAL, new_nodes, 0.0)
        nodes[...] = new_nodes

        if t < STEPS - 1:
            abuf[...] = (_mm(_relu(_mm(new_nodes, w_ps[...]) + b_ps[...]),
                             w_ur[...]) * 0.5 + b_ur[...])
            bbuf[...] = _mm(_relu(_mm(new_nodes, w_po[...]) + b_po[...]),
                            w_ur[...]) * 0.5
            a_full = abuf[...]
            b_full = bbuf[...]
            dg = _relu(dg + a_full + b_full)
            pc = _relu(pc + a_full + b_full[TOTAL:TOTAL + 1, :])
            pr = _relu(pr + a_full[TOTAL:TOTAL + 1, :] + b_full)
            acc_o[...] = jnp.zeros((NP, D), f32)
            last = t == STEPS - 2

            def sweep(c, _):
                base = c * SC
                blk = rels[pl.ds(base, SC)]
                a3 = abuf[pl.ds(base, SC), :].reshape(SC, 1, D)
                b3 = bbuf[...].reshape(1, NP, D)
                new = _relu(blk + a3 + b3)
                if not last:
                    rels[pl.ds(base, SC)] = new
                acc_s[pl.ds(base, SC), :] = jnp.sum(new, axis=1)
                acc_o[...] += jnp.sum(new, axis=0)
                return 0

            jax.lax.fori_loop(0, NP // SC, sweep, 0)

    obj = nodes[0:OBJ_N, :]

    def _dec(w1h, b1h, w2h, b2h):
        hh = _mm(obj, w1h[...]) + b1h[...]
        hh = jnp.where(hh > 0, hh, 0.2 * hh)
        return _mm(hh, w2h[...]) + b2h[...]

    out[:, 0:3] = _dec(w_t1, b_t1, w_t2, b_t2)
    out[:, 3:6] = _dec(w_e1, b_e1, w_e2, b_e2)
    out[:, 6:9] = _dec(w_s1, b_s1, w_s2, b_s2)


def kernel(trans_object_obb, trans_object_abb, trans_object_obb_center,
           trans_object_obb_center_dist, trans_object_abb_eiou,
           wall_position, wall_normal, floor_position, floor_normal,
           floor_z_value, params):
    f32 = jnp.float32
    p = params
    args = [trans_object_obb.reshape(OBJ_N, 24),
            trans_object_abb.reshape(OBJ_N, 6),
            trans_object_obb_center.reshape(OBJ_N, 3),
            trans_object_obb_center_dist.reshape(OBJ_N, OBJ_N),
            trans_object_abb_eiou.reshape(OBJ_N, OBJ_N),
            wall_position.reshape(WALL_N, 12),
            wall_normal.reshape(WALL_N, 3),
            floor_position.reshape(1, 12),
            floor_normal.reshape(1, 3),
            floor_z_value.reshape(1, 1)]
    for name in ['obj1', 'obj2', 'rel1', 'rel2', 'wal1', 'wal2',
                 'flr1', 'flr2', 'nn', 'sp', 'op', 'un', 'ps', 'po', 'ur',
                 't1', 't2', 'e1', 'e2', 's1', 's2']:
        args.append(p[name][0])
        args.append(p[name][1].reshape(1, -1))

    out = pl.pallas_call(
        _body,
        out_shape=jax.ShapeDtypeStruct((OBJ_N, 9), f32),
        scratch_shapes=[
            pltpu.VMEM((NP, NP, D), f32),
            pltpu.VMEM((NP, D), f32),
            pltpu.VMEM((NP, D), f32),
            pltpu.VMEM((NP, D), f32),
            pltpu.VMEM((NP, D), f32),
            pltpu.VMEM((NP, D), f32),
            pltpu.VMEM((D, D), jnp.bfloat16),
        ],
        compiler_params=pltpu.CompilerParams(
            vmem_limit_bytes=100 * 1024 * 1024),
    )(*args)
    return out


# speedup vs baseline: 1.5019x; 1.0301x over previous
import jax
import jax.numpy as jnp
from jax.experimental import pallas as pl
from jax.experimental.pallas import tpu as pltpu

OBJ_N = 128
WALL_N = 8
TOTAL = 137
NP = 144
D = 512
STEPS = 4
INV_DEG = 1.0 / float(TOTAL - 1)


def _mm(x, w):
    return jax.lax.dot_general(x, w, (((1,), (0,)), ((), ())),
                               preferred_element_type=jnp.float32)


def _relu(x):
    return jnp.maximum(x, 0.0)


def _body(obb, abb, ctr, rel_a, rel_b, wpos, wnrm, fpos, fnrm, fz,
          w_obj1, b_obj1, w_obj2, b_obj2,
          w_rel1, b_rel1, w_rel2, b_rel2,
          w_wal1, b_wal1, w_wal2, b_wal2,
          w_flr1, b_flr1, w_flr2, b_flr2,
          w_nn, b_nn, w_sp, b_sp, w_op, b_op, w_un, b_un,
          w_ps, b_ps, w_po, b_po, w_ur, b_ur,
          w_t1, b_t1, w_t2, b_t2,
          w_e1, b_e1, w_e2, b_e2,
          w_s1, b_s1, w_s2, b_s2,
          out,
          rels, nodes, acc_s, acc_o, abuf, bbuf, wrel2b):
    f32 = jnp.float32

    obj_x = jnp.concatenate([obb[...], abb[...], ctr[...]], axis=1)
    h = _relu(_mm(obj_x, w_obj1[...]) + b_obj1[...])
    obj_emb = _mm(h, w_obj2[...]) + b_obj2[...]
    wall_x = jnp.concatenate([wpos[...], wnrm[...]], axis=1)
    h = _relu(_mm(wall_x, w_wal1[...]) + b_wal1[...])
    wall_emb = _mm(h, w_wal2[...]) + b_wal2[...]
    floor_x = jnp.concatenate([fpos[...], fnrm[...], fz[...]], axis=1)
    h = _relu(_mm(floor_x, w_flr1[...]) + b_flr1[...])
    floor_emb = _mm(h, w_flr2[...]) + b_flr2[...]
    nodes[0:OBJ_N, :] = obj_emb
    nodes[OBJ_N:OBJ_N + WALL_N, :] = wall_emb
    nodes[TOTAL - 1:TOTAL, :] = floor_emb
    nodes[TOTAL:NP, :] = jnp.zeros((NP - TOTAL, D), f32)

    acc_o[...] = jnp.zeros((NP, D), f32)
    acc_s[...] = jnp.zeros((NP, D), f32)
    wrel2b[...] = w_rel2[...].astype(jnp.bfloat16)

    rat = jnp.transpose(rel_a[...], (1, 0))
    rbt = jnp.transpose(rel_b[...], (1, 0))
    w0 = w_rel1[0:1, :]
    w1 = w_rel1[1:2, :]
    br1 = b_rel1[...]
    br2 = b_rel2[...]

    BC = 8
    ohbase = (jax.lax.broadcasted_iota(jnp.int32, (OBJ_N, BC), 0)
              - jax.lax.broadcasted_iota(jnp.int32, (OBJ_N, BC), 1))
    subj_m = jax.lax.broadcasted_iota(jnp.int32, (BC, OBJ_N, D), 0)
    j_m = jax.lax.broadcasted_iota(jnp.int32, (BC, OBJ_N, D), 1)
    subj_t = jax.lax.broadcasted_iota(jnp.int32, (BC, NP - OBJ_N, D), 0)
    j_t = jax.lax.broadcasted_iota(jnp.int32, (BC, NP - OBJ_N, D), 1) + OBJ_N

    def build(c, _):
        base = c * BC
        oh = (ohbase == base).astype(f32)
        a_g = _mm(rat, oh)
        b_g = _mm(rbt, oh)
        a_st = jnp.concatenate([a_g[:, m:m + 1] for m in range(BC)], axis=0)
        b_st = jnp.concatenate([b_g[:, m:m + 1] for m in range(BC)], axis=0)
        hh = _relu(a_st * w0 + b_st * w1 + br1)
        emb = _mm(hh.astype(jnp.bfloat16), wrel2b[...]) + br2
        emb3 = emb.reshape(BC, OBJ_N, D)
        sm = subj_m + base
        st = subj_t + base
        main3 = jnp.where((j_m != sm) & (sm < TOTAL),
                          jnp.where(sm < OBJ_N, emb3, 0.001), 0.0)
        tail3 = jnp.where((j_t < TOTAL) & (j_t != st) & (st < TOTAL),
                          0.001, 0.0)
        full3 = jnp.concatenate([main3, tail3], axis=1)
        rels[pl.ds(base, BC)] = full3
        acc_s[pl.ds(base, BC), :] = jnp.sum(full3, axis=1)
        acc_o[...] += jnp.sum(full3, axis=0)
        return 0

    jax.lax.fori_loop(0, OBJ_N // BC, build, 0)

    ts = jax.lax.broadcasted_iota(jnp.int32, (NP - OBJ_N, NP, D), 0) + OBJ_N
    tj = jax.lax.broadcasted_iota(jnp.int32, (NP - OBJ_N, NP, D), 1)
    tail16 = jnp.where((tj < TOTAL) & (tj != ts) & (ts < TOTAL), 0.001, 0.0)
    rels[pl.ds(OBJ_N, NP - OBJ_N)] = tail16
    acc_s[pl.ds(OBJ_N, NP - OBJ_N), :] = jnp.sum(tail16, axis=1)
    acc_o[...] += jnp.sum(tail16, axis=0)

    row_iota = jax.lax.broadcasted_iota(jnp.int32, (NP, D), 0)
    SC = 16
    dg = jnp.zeros((NP, D), f32)
    pc = jnp.zeros((NP, D), f32)
    pr = jnp.zeros((NP, D), f32)
    for t in range(STEPS):
        nodes_v = nodes[...]
        mean = jnp.sum(nodes_v, axis=0, keepdims=True) / float(TOTAL)
        c_nn = _relu(_mm(mean, w_nn[...]) + b_nn[...])
        agg_s = (acc_s[...] - dg - 7.0 * pc) * INV_DEG
        agg_o = (acc_o[...] - dg - 7.0 * pr) * INV_DEG
        c_sp = _relu(_mm(agg_s, w_sp[...]) + b_sp[...])
        c_op = _relu(_mm(agg_o, w_op[...]) + b_op[...])
        c = (c_nn + c_sp + c_op) / 3.0
        new_nodes = _relu(nodes_v + _mm(c, w_un[...]) + b_un[...])
        new_nodes = jnp.where(row_iota < TOTAL, new_nodes, 0.0)
        nodes[...] = new_nodes

        if t < STEPS - 1:
            abuf[...] = (_mm(_relu(_mm(new_nodes, w_ps[...]) + b_ps[...]),
                             w_ur[...]) * 0.5 + b_ur[...])
            bbuf[...] = _mm(_relu(_mm(new_nodes, w_po[...]) + b_po[...]),
                            w_ur[...]) * 0.5
            a_full = abuf[...]
            b_full = bbuf[...]
            dg = _relu(dg + a_full + b_full)
            pc = _relu(pc + a_full + b_full[TOTAL:TOTAL + 1, :])
            pr = _relu(pr + a_full[TOTAL:TOTAL + 1, :] + b_full)
            acc_o[...] = jnp.zeros((NP, D), f32)
            last = t == STEPS - 2

            def sweep(c, _):
                base = c * SC
                blk = rels[pl.ds(base, SC)]
                a3 = abuf[pl.ds(base, SC), :].reshape(SC, 1, D)
                b3 = bbuf[...].reshape(1, NP, D)
                new = _relu(blk + a3 + b3)
                if not last:
                    rels[pl.ds(base, SC)] = new
                acc_s[pl.ds(base, SC), :] = jnp.sum(new, axis=1)
                acc_o[...] += jnp.sum(new, axis=0)
                return 0

            jax.lax.fori_loop(0, NP // SC, sweep, 0)

    obj = nodes[0:OBJ_N, :]

    def _dec(w1h, b1h, w2h, b2h):
        hh = _mm(obj, w1h[...]) + b1h[...]
        hh = jnp.where(hh > 0, hh, 0.2 * hh)
        return _mm(hh, w2h[...]) + b2h[...]

    out[:, 0:3] = _dec(w_t1, b_t1, w_t2, b_t2)
    out[:, 3:6] = _dec(w_e1, b_e1, w_e2, b_e2)
    out[:, 6:9] = _dec(w_s1, b_s1, w_s2, b_s2)


def kernel(trans_object_obb, trans_object_abb, trans_object_obb_center,
           trans_object_obb_center_dist, trans_object_abb_eiou,
           wall_position, wall_normal, floor_position, floor_normal,
           floor_z_value, params):
    f32 = jnp.float32
    p = params
    args = [trans_object_obb.reshape(OBJ_N, 24),
            trans_object_abb.reshape(OBJ_N, 6),
            trans_object_obb_center.reshape(OBJ_N, 3),
            trans_object_obb_center_dist.reshape(OBJ_N, OBJ_N),
            trans_object_abb_eiou.reshape(OBJ_N, OBJ_N),
            wall_position.reshape(WALL_N, 12),
            wall_normal.reshape(WALL_N, 3),
            floor_position.reshape(1, 12),
            floor_normal.reshape(1, 3),
            floor_z_value.reshape(1, 1)]
    for name in ['obj1', 'obj2', 'rel1', 'rel2', 'wal1', 'wal2',
                 'flr1', 'flr2', 'nn', 'sp', 'op', 'un', 'ps', 'po', 'ur',
                 't1', 't2', 'e1', 'e2', 's1', 's2']:
        args.append(p[name][0])
        args.append(p[name][1].reshape(1, -1))

    out = pl.pallas_call(
        _body,
        out_shape=jax.ShapeDtypeStruct((OBJ_N, 9), f32),
        scratch_shapes=[
            pltpu.VMEM((NP, NP, D), f32),
            pltpu.VMEM((NP, D), f32),
            pltpu.VMEM((NP, D), f32),
            pltpu.VMEM((NP, D), f32),
            pltpu.VMEM((NP, D), f32),
            pltpu.VMEM((NP, D), f32),
            pltpu.VMEM((D, D), jnp.bfloat16),
        ],
        compiler_params=pltpu.CompilerParams(
            vmem_limit_bytes=100 * 1024 * 1024),
    )(*args)
    return out
